# baseline (speedup 1.0000x reference)
import jax
import jax.numpy as jnp
from jax.experimental import pallas as pl
from jax.experimental.pallas import tpu as pltpu

_CLIP = 5.0


def _enc_body(x_ref, wx_ref, bx_ref, wn_ref,
              ci_ref, hn_ref, h_ref, cibuf_ref, sems, *, kt, nt, b, h):
    i = pl.program_id(0)
    t = pl.program_id(1)
    tt = jnp.where(i == 0, t, nt - 1 - t)
    row0 = i * b

    @pl.when(t == 0)
    def _():
        h_ref[...] = jnp.zeros_like(h_ref)

    wx = wx_ref[0]
    bx = bx_ref[0]
    wn = wn_ref[0]

    def substep(j):
        hc = h_ref[...]
        cibuf_ref[j] = hc[:, h:]
        pltpu.make_async_copy(
            cibuf_ref.at[j],
            ci_ref.at[pl.ds(row0, b), tt * kt + j, :],
            sems.at[j],
        ).start()
        x = x_ref[:, j, :]
        pre = jnp.dot(jnp.concatenate([x, hc], axis=1), wx,
                      preferred_element_type=jnp.float32) + bx
        z = 0.5 * jnp.tanh(pre[:, : 2 * h]) + 0.5
        r = 0.5 * jnp.tanh(pre[:, 2 * h : 4 * h]) + 0.5
        npre = jnp.dot(r * hc, wn, preferred_element_type=jnp.float32)
        n = jnp.tanh(pre[:, 4 * h :] + npre)
        hnew = n + z * (hc - n)
        h_ref[...] = jnp.clip(hnew, -_CLIP, _CLIP)

    @pl.when(i == 0)
    def _():
        for j in range(kt):
            substep(j)

    @pl.when(i == 1)
    def _():
        for j in range(kt - 1, -1, -1):
            substep(j)

    for j in range(kt):
        pltpu.make_async_copy(
            cibuf_ref.at[j],
            ci_ref.at[pl.ds(row0, b), tt * kt + j, :],
            sems.at[j],
        ).wait()

    @pl.when(t == nt - 1)
    def _():
        hn_ref[0] = h_ref[...][:, :h]


def _head_body(hn_ref, wl_ref, bl_ref, mean_ref, std_ref, *, ic_dim):
    hcat = jnp.concatenate([hn_ref[0], hn_ref[1]], axis=-1)
    params = jnp.dot(hcat, wl_ref[...], preferred_element_type=jnp.float32)
    params = params + bl_ref[...]
    mean_ref[...] = params[:, :ic_dim]
    std_ref[...] = jnp.exp(params[:, ic_dim:])


def kernel(data, ic_h0, ci_h0, ic_Wih, ic_bih, ic_Whh, ic_bhh,
           ci_Wih, ci_bih, ci_Whh, ci_bhh, Wl, bl):
    b, t_len, d = data.shape
    h = ic_Whh.shape[-1]
    ic_dim = Wl.shape[0] // 2
    kt = 8 if t_len % 8 == 0 else 1
    nt = t_len // kt

    def colsT(w, lo, hi):
        return jnp.swapaxes(w[:, lo:hi, :], 1, 2)

    wx = jnp.concatenate(
        [colsT(ic_Wih, 0, h), colsT(ci_Wih, 0, h),
         colsT(ic_Wih, h, 2 * h), colsT(ci_Wih, h, 2 * h),
         colsT(ic_Wih, 2 * h, 3 * h), colsT(ci_Wih, 2 * h, 3 * h)],
        axis=2)
    bx = jnp.concatenate(
        [ic_bih[:, :h], ci_bih[:, :h],
         ic_bih[:, h:2 * h], ci_bih[:, h:2 * h],
         ic_bih[:, 2 * h:], ci_bih[:, 2 * h:]], axis=1)[:, None, :]

    wzr = jnp.zeros((2, 2 * h, 4 * h), jnp.float32)
    wzr = wzr.at[:, :h, 0:h].set(colsT(ic_Whh, 0, h))
    wzr = wzr.at[:, h:, h:2 * h].set(colsT(ci_Whh, 0, h))
    wzr = wzr.at[:, :h, 2 * h:3 * h].set(colsT(ic_Whh, h, 2 * h))
    wzr = wzr.at[:, h:, 3 * h:].set(colsT(ci_Whh, h, 2 * h))
    bzr = jnp.concatenate(
        [ic_bhh[:, :h], ci_bhh[:, :h],
         ic_bhh[:, h:2 * h], ci_bhh[:, h:2 * h]], axis=1)[:, None, :]

    wn = jnp.zeros((2, 2 * h, 2 * h), jnp.float32)
    wn = wn.at[:, :h, :h].set(colsT(ic_Whh, 2 * h, 3 * h))
    wn = wn.at[:, h:, h:].set(colsT(ci_Whh, 2 * h, 3 * h))
    bn = jnp.concatenate([ic_bhh[:, 2 * h:], ci_bhh[:, 2 * h:]],
                         axis=1)[:, None, :]
    bx = bx + jnp.concatenate([bzr, bn], axis=2)
    bx = bx.at[:, :, : 4 * h].multiply(0.5)
    wx = wx.at[:, :, : 4 * h].multiply(0.5)
    wzr = wzr * 0.5
    wx = jnp.concatenate(
        [wx, jnp.pad(wzr, ((0, 0), (0, 0), (0, 2 * h)))], axis=1)

    def tmap(i, tt):
        return (0, jnp.where(i == 0, tt, nt - 1 - tt), 0)

    from functools import partial
    ci_out, hn = pl.pallas_call(
        partial(_enc_body, kt=kt, nt=nt, b=b, h=h),
        grid=(2, nt),
        in_specs=[
            pl.BlockSpec((b, kt, d), tmap),
            pl.BlockSpec((1, d + 2 * h, 6 * h), lambda i, tt: (i, 0, 0)),
            pl.BlockSpec((1, 1, 6 * h), lambda i, tt: (i, 0, 0)),
            pl.BlockSpec((1, 2 * h, 2 * h), lambda i, tt: (i, 0, 0)),
        ],
        out_specs=[
            pl.BlockSpec(memory_space=pl.ANY),
            pl.BlockSpec((1, b, h), lambda i, tt: (i, 0, 0)),
        ],
        out_shape=[
            jax.ShapeDtypeStruct((2 * b, t_len, h), jnp.float32),
            jax.ShapeDtypeStruct((2, b, h), jnp.float32),
        ],
        scratch_shapes=[
            pltpu.VMEM((b, 2 * h), jnp.float32),
            pltpu.VMEM((kt, b, h), jnp.float32),
            pltpu.SemaphoreType.DMA((kt,)),
        ],
        compiler_params=pltpu.CompilerParams(
            dimension_semantics=("parallel", "arbitrary"),
        ),
        name="bigru_encoder",
    )(data, wx, bx, wn)

    ic_mean, ic_std = pl.pallas_call(
        partial(_head_body, ic_dim=ic_dim),
        out_shape=[
            jax.ShapeDtypeStruct((b, ic_dim), jnp.float32),
            jax.ShapeDtypeStruct((b, ic_dim), jnp.float32),
        ],
        name="ic_head",
    )(hn, Wl.T, bl[None, :])

    return ic_mean, ic_std, ci_out


# speedup vs baseline: 1.0871x; 1.0871x over previous
import jax
import jax.numpy as jnp
from jax.experimental import pallas as pl
from jax.experimental.pallas import tpu as pltpu

_CLIP = 5.0


def _enc_body(x_ref, wx_ref, bx_ref, wzr_ref, wn_ref,
              ci_ref, hn_ref, h_ref, cibuf_ref, sems,
              xbuf_ref, xsems, *, kt, nt, b, h):
    i = pl.program_id(0)
    t = pl.program_id(1)
    tt = jnp.where(i == 0, t, nt - 1 - t)
    row0 = i * b

    def start_x_dmas(slot, blk):
        for j in range(kt):
            pltpu.make_async_copy(
                x_ref.at[:, blk * kt + j, :],
                xbuf_ref.at[slot, j],
                xsems.at[slot, j],
            ).start()

    @pl.when(t == 0)
    def _():
        h_ref[...] = jnp.zeros_like(h_ref)
        start_x_dmas(0, tt)

    @pl.when(t + 1 < nt)
    def _():
        tt_next = jnp.where(i == 0, t + 1, nt - 2 - t)
        start_x_dmas((t + 1) % 2, tt_next)

    slot = t % 2
    for j in range(kt):
        pltpu.make_async_copy(
            x_ref.at[:, 0, :],
            xbuf_ref.at[slot, j],
            xsems.at[slot, j],
        ).wait()

    wx = wx_ref[0]
    bx = bx_ref[0]
    wzr = wzr_ref[0]
    wn = wn_ref[0]

    def substep(j):
        hc = h_ref[...]
        cibuf_ref[j] = hc[:, h:]
        pltpu.make_async_copy(
            cibuf_ref.at[j],
            ci_ref.at[pl.ds(row0, b), tt * kt + j, :],
            sems.at[j],
        ).start()
        x = xbuf_ref[slot, j]
        xa = jnp.dot(x, wx, preferred_element_type=jnp.float32) + bx
        hzr = jnp.dot(hc, wzr, preferred_element_type=jnp.float32)
        z = 0.5 * jnp.tanh(xa[:, : 2 * h] + hzr[:, : 2 * h]) + 0.5
        thr = jnp.tanh(xa[:, 2 * h : 4 * h] + hzr[:, 2 * h :])
        npre = jnp.dot(thr * hc + hc, wn, preferred_element_type=jnp.float32)
        n = jnp.tanh(xa[:, 4 * h :] + npre)
        hnew = n + z * (hc - n)
        h_ref[...] = jnp.clip(hnew, -_CLIP, _CLIP)

    @pl.when(i == 0)
    def _():
        for j in range(kt):
            substep(j)

    @pl.when(i == 1)
    def _():
        for j in range(kt - 1, -1, -1):
            substep(j)

    for j in range(kt):
        pltpu.make_async_copy(
            cibuf_ref.at[j],
            ci_ref.at[pl.ds(row0, b), tt * kt + j, :],
            sems.at[j],
        ).wait()

    @pl.when(t == nt - 1)
    def _():
        hn_ref[0] = h_ref[...][:, :h]


def _head_body(hn_ref, wl_ref, bl_ref, mean_ref, std_ref, *, ic_dim):
    hcat = jnp.concatenate([hn_ref[0], hn_ref[1]], axis=-1)
    params = jnp.dot(hcat, wl_ref[...], preferred_element_type=jnp.float32)
    params = params + bl_ref[...]
    mean_ref[...] = params[:, :ic_dim]
    std_ref[...] = jnp.exp(params[:, ic_dim:])


def kernel(data, ic_h0, ci_h0, ic_Wih, ic_bih, ic_Whh, ic_bhh,
           ci_Wih, ci_bih, ci_Whh, ci_bhh, Wl, bl):
    b, t_len, d = data.shape
    h = ic_Whh.shape[-1]
    ic_dim = Wl.shape[0] // 2
    kt = 8 if t_len % 8 == 0 else 1
    nt = t_len // kt

    def colsT(w, lo, hi):
        return jnp.swapaxes(w[:, lo:hi, :], 1, 2)

    wx = jnp.concatenate(
        [colsT(ic_Wih, 0, h), colsT(ci_Wih, 0, h),
         colsT(ic_Wih, h, 2 * h), colsT(ci_Wih, h, 2 * h),
         colsT(ic_Wih, 2 * h, 3 * h), colsT(ci_Wih, 2 * h, 3 * h)],
        axis=2)
    bx = jnp.concatenate(
        [ic_bih[:, :h], ci_bih[:, :h],
         ic_bih[:, h:2 * h], ci_bih[:, h:2 * h],
         ic_bih[:, 2 * h:], ci_bih[:, 2 * h:]], axis=1)[:, None, :]

    wzr = jnp.zeros((2, 2 * h, 4 * h), jnp.float32)
    wzr = wzr.at[:, :h, 0:h].set(colsT(ic_Whh, 0, h))
    wzr = wzr.at[:, h:, h:2 * h].set(colsT(ci_Whh, 0, h))
    wzr = wzr.at[:, :h, 2 * h:3 * h].set(colsT(ic_Whh, h, 2 * h))
    wzr = wzr.at[:, h:, 3 * h:].set(colsT(ci_Whh, h, 2 * h))
    bzr = jnp.concatenate(
        [ic_bhh[:, :h], ci_bhh[:, :h],
         ic_bhh[:, h:2 * h], ci_bhh[:, h:2 * h]], axis=1)[:, None, :]

    wn = jnp.zeros((2, 2 * h, 2 * h), jnp.float32)
    wn = wn.at[:, :h, :h].set(colsT(ic_Whh, 2 * h, 3 * h))
    wn = wn.at[:, h:, h:].set(colsT(ci_Whh, 2 * h, 3 * h))
    bn = jnp.concatenate([ic_bhh[:, 2 * h:], ci_bhh[:, 2 * h:]],
                         axis=1)[:, None, :]
    bx = bx + jnp.concatenate([bzr, bn], axis=2)
    bx = bx.at[:, :, : 4 * h].multiply(0.5)
    wx = wx.at[:, :, : 4 * h].multiply(0.5)
    wzr = wzr * 0.5
    wn = wn * 0.5

    from functools import partial
    ci_out, hn = pl.pallas_call(
        partial(_enc_body, kt=kt, nt=nt, b=b, h=h),
        grid=(2, nt),
        in_specs=[
            pl.BlockSpec(memory_space=pl.ANY),
            pl.BlockSpec((1, d, 6 * h), lambda i, tt: (i, 0, 0)),
            pl.BlockSpec((1, 1, 6 * h), lambda i, tt: (i, 0, 0)),
            pl.BlockSpec((1, 2 * h, 4 * h), lambda i, tt: (i, 0, 0)),
            pl.BlockSpec((1, 2 * h, 2 * h), lambda i, tt: (i, 0, 0)),
        ],
        out_specs=[
            pl.BlockSpec(memory_space=pl.ANY),
            pl.BlockSpec((1, b, h), lambda i, tt: (i, 0, 0)),
        ],
        out_shape=[
            jax.ShapeDtypeStruct((2 * b, t_len, h), jnp.float32),
            jax.ShapeDtypeStruct((2, b, h), jnp.float32),
        ],
        scratch_shapes=[
            pltpu.VMEM((b, 2 * h), jnp.float32),
            pltpu.VMEM((kt, b, h), jnp.float32),
            pltpu.SemaphoreType.DMA((kt,)),
            pltpu.VMEM((2, kt, b, d), jnp.float32),
            pltpu.SemaphoreType.DMA((2, kt)),
        ],
        compiler_params=pltpu.CompilerParams(
            dimension_semantics=("parallel", "arbitrary"),
        ),
        name="bigru_encoder",
    )(data, wx, bx, wzr, wn)

    ic_mean, ic_std = pl.pallas_call(
        partial(_head_body, ic_dim=ic_dim),
        out_shape=[
            jax.ShapeDtypeStruct((b, ic_dim), jnp.float32),
            jax.ShapeDtypeStruct((b, ic_dim), jnp.float32),
        ],
        name="ic_head",
    )(hn, Wl.T, bl[None, :])

    return ic_mean, ic_std, ci_out


# speedup vs baseline: 1.1196x; 1.0299x over previous
import jax
import jax.numpy as jnp
from jax.experimental import pallas as pl
from jax.experimental.pallas import tpu as pltpu

_CLIP = 5.0


def _enc_body(x_ref, wx_ref, bx_ref, wzr_ref, wn_ref,
              ci_ref, hn_ref, h_ref, cibuf_ref, sems,
              xbuf_ref, xsems, *, kt, nt, b, h):
    i = pl.program_id(0)
    t = pl.program_id(1)
    tt = jnp.where(i == 0, t, nt - 1 - t)
    row0 = i * b

    def start_x_dmas(slot, blk):
        for j in range(kt):
            pltpu.make_async_copy(
                x_ref.at[:, blk * kt + j, :],
                xbuf_ref.at[slot, j],
                xsems.at[slot, j],
            ).start()

    @pl.when(t == 0)
    def _():
        h_ref[...] = jnp.zeros_like(h_ref)
        start_x_dmas(0, tt)

    @pl.when(t + 1 < nt)
    def _():
        tt_next = jnp.where(i == 0, t + 1, nt - 2 - t)
        start_x_dmas((t + 1) % 2, tt_next)

    slot = t % 2
    for j in range(kt):
        pltpu.make_async_copy(
            x_ref.at[:, 0, :],
            xbuf_ref.at[slot, j],
            xsems.at[slot, j],
        ).wait()

    wx = wx_ref[0]
    bx = bx_ref[0]
    wzr = wzr_ref[0]
    wn = wn_ref[0]

    b2 = b // 2

    def substep(j):
        for half in range(2):
            rows = pl.ds(half * b2, b2)
            hc = h_ref[rows, :]
            cibuf_ref[j, rows, :] = hc[:, h:]
            x = xbuf_ref[slot, j, rows, :]
            xa = jnp.dot(x, wx, preferred_element_type=jnp.float32) + bx
            hzr = jnp.dot(hc, wzr, preferred_element_type=jnp.float32)
            z = 0.5 * jnp.tanh(xa[:, : 2 * h] + hzr[:, : 2 * h]) + 0.5
            thr = jnp.tanh(xa[:, 2 * h : 4 * h] + hzr[:, 2 * h :])
            npre = jnp.dot(thr * hc + hc, wn,
                           preferred_element_type=jnp.float32)
            n = jnp.tanh(xa[:, 4 * h :] + npre)
            hnew = n + z * (hc - n)
            h_ref[rows, :] = jnp.clip(hnew, -_CLIP, _CLIP)
        pltpu.make_async_copy(
            cibuf_ref.at[j],
            ci_ref.at[pl.ds(row0, b), tt * kt + j, :],
            sems.at[j],
        ).start()

    @pl.when(i == 0)
    def _():
        for j in range(kt):
            substep(j)

    @pl.when(i == 1)
    def _():
        for j in range(kt - 1, -1, -1):
            substep(j)

    for j in range(kt):
        pltpu.make_async_copy(
            cibuf_ref.at[j],
            ci_ref.at[pl.ds(row0, b), tt * kt + j, :],
            sems.at[j],
        ).wait()

    @pl.when(t == nt - 1)
    def _():
        hn_ref[0] = h_ref[...][:, :h]


def _head_body(hn_ref, wl_ref, bl_ref, mean_ref, std_ref, *, ic_dim):
    hcat = jnp.concatenate([hn_ref[0], hn_ref[1]], axis=-1)
    params = jnp.dot(hcat, wl_ref[...], preferred_element_type=jnp.float32)
    params = params + bl_ref[...]
    mean_ref[...] = params[:, :ic_dim]
    std_ref[...] = jnp.exp(params[:, ic_dim:])


def kernel(data, ic_h0, ci_h0, ic_Wih, ic_bih, ic_Whh, ic_bhh,
           ci_Wih, ci_bih, ci_Whh, ci_bhh, Wl, bl):
    b, t_len, d = data.shape
    h = ic_Whh.shape[-1]
    ic_dim = Wl.shape[0] // 2
    kt = 8 if t_len % 8 == 0 else 1
    nt = t_len // kt

    def colsT(w, lo, hi):
        return jnp.swapaxes(w[:, lo:hi, :], 1, 2)

    wx = jnp.concatenate(
        [colsT(ic_Wih, 0, h), colsT(ci_Wih, 0, h),
         colsT(ic_Wih, h, 2 * h), colsT(ci_Wih, h, 2 * h),
         colsT(ic_Wih, 2 * h, 3 * h), colsT(ci_Wih, 2 * h, 3 * h)],
        axis=2)
    bx = jnp.concatenate(
        [ic_bih[:, :h], ci_bih[:, :h],
         ic_bih[:, h:2 * h], ci_bih[:, h:2 * h],
         ic_bih[:, 2 * h:], ci_bih[:, 2 * h:]], axis=1)[:, None, :]

    wzr = jnp.zeros((2, 2 * h, 4 * h), jnp.float32)
    wzr = wzr.at[:, :h, 0:h].set(colsT(ic_Whh, 0, h))
    wzr = wzr.at[:, h:, h:2 * h].set(colsT(ci_Whh, 0, h))
    wzr = wzr.at[:, :h, 2 * h:3 * h].set(colsT(ic_Whh, h, 2 * h))
    wzr = wzr.at[:, h:, 3 * h:].set(colsT(ci_Whh, h, 2 * h))
    bzr = jnp.concatenate(
        [ic_bhh[:, :h], ci_bhh[:, :h],
         ic_bhh[:, h:2 * h], ci_bhh[:, h:2 * h]], axis=1)[:, None, :]

    wn = jnp.zeros((2, 2 * h, 2 * h), jnp.float32)
    wn = wn.at[:, :h, :h].set(colsT(ic_Whh, 2 * h, 3 * h))
    wn = wn.at[:, h:, h:].set(colsT(ci_Whh, 2 * h, 3 * h))
    bn = jnp.concatenate([ic_bhh[:, 2 * h:], ci_bhh[:, 2 * h:]],
                         axis=1)[:, None, :]
    bx = bx + jnp.concatenate([bzr, bn], axis=2)
    bx = bx.at[:, :, : 4 * h].multiply(0.5)
    wx = wx.at[:, :, : 4 * h].multiply(0.5)
    wzr = wzr * 0.5
    wn = wn * 0.5

    from functools import partial
    ci_out, hn = pl.pallas_call(
        partial(_enc_body, kt=kt, nt=nt, b=b, h=h),
        grid=(2, nt),
        in_specs=[
            pl.BlockSpec(memory_space=pl.ANY),
            pl.BlockSpec((1, d, 6 * h), lambda i, tt: (i, 0, 0)),
            pl.BlockSpec((1, 1, 6 * h), lambda i, tt: (i, 0, 0)),
            pl.BlockSpec((1, 2 * h, 4 * h), lambda i, tt: (i, 0, 0)),
            pl.BlockSpec((1, 2 * h, 2 * h), lambda i, tt: (i, 0, 0)),
        ],
        out_specs=[
            pl.BlockSpec(memory_space=pl.ANY),
            pl.BlockSpec((1, b, h), lambda i, tt: (i, 0, 0)),
        ],
        out_shape=[
            jax.ShapeDtypeStruct((2 * b, t_len, h), jnp.float32),
            jax.ShapeDtypeStruct((2, b, h), jnp.float32),
        ],
        scratch_shapes=[
            pltpu.VMEM((b, 2 * h), jnp.float32),
            pltpu.VMEM((kt, b, h), jnp.float32),
            pltpu.SemaphoreType.DMA((kt,)),
            pltpu.VMEM((2, kt, b, d), jnp.float32),
            pltpu.SemaphoreType.DMA((2, kt)),
        ],
        compiler_params=pltpu.CompilerParams(
            dimension_semantics=("parallel", "arbitrary"),
        ),
        name="bigru_encoder",
    )(data, wx, bx, wzr, wn)

    ic_mean, ic_std = pl.pallas_call(
        partial(_head_body, ic_dim=ic_dim),
        out_shape=[
            jax.ShapeDtypeStruct((b, ic_dim), jnp.float32),
            jax.ShapeDtypeStruct((b, ic_dim), jnp.float32),
        ],
        name="ic_head",
    )(hn, Wl.T, bl[None, :])

    return ic_mean, ic_std, ci_out
